# Initial kernel scaffold; baseline (speedup 1.0000x reference)
import jax
import jax.numpy as jnp
from jax.experimental import pallas as pl


def kernel(input_seqs, seq_lengths, table, W_ih, W_hh, b_ih, b_hh):
    raise NotImplementedError("write your pallas kernel here")


# speedup vs baseline: 2.8655x; 2.8655x over previous
import functools

import jax
import jax.numpy as jnp
from jax import lax
from jax.experimental import pallas as pl
from jax.experimental.pallas import tpu as pltpu
from jax.experimental.pallas import tpu_sc as plsc


def _make_sc_gather(V, E, N):
    info = plsc.get_sparse_core_info()
    NW = info.num_cores * info.num_subcores
    NC = info.num_cores
    per_w = N // NW
    CH = 80
    n_ch = per_w // CH
    assert per_w * NW == N and n_ch * CH == per_w

    mesh = plsc.VectorSubcoreMesh(core_axis_name="c", subcore_axis_name="s")

    @functools.partial(
        pl.kernel,
        mesh=mesh,
        out_type=jax.ShapeDtypeStruct((N, E), jnp.float32),
        scratch_types=[
            pltpu.VMEM((n_ch, CH), jnp.int32),
            pltpu.VMEM((per_w, E), jnp.float32),
            pltpu.SemaphoreType.DMA,
        ],
        compiler_params=pltpu.CompilerParams(use_tc_tiling_on_sc=False),
    )
    def gather_k(table_hbm, idx_hbm, out_hbm, idx_v, rows_v, sem):
        wid = lax.axis_index("s") * NC + lax.axis_index("c")
        base = wid * per_w
        pltpu.sync_copy(idx_hbm.at[wid], idx_v)
        copies = []
        for j in range(n_ch):
            copies.append(
                pltpu.async_copy(
                    table_hbm.at[idx_v.at[j]],
                    rows_v.at[pl.ds(j * CH, CH)],
                    sem,
                )
            )
        for c in copies:
            c.wait()
        pltpu.sync_copy(rows_v, out_hbm.at[pl.ds(base, per_w)])

    return gather_k


def _gru_body(L, H, lens_ref, wih_ref, whh_ref, bih_ref, bhh_ref, x_ref,
              out_ref, hid_ref, h_scr):
    t = pl.program_id(0)

    @pl.when(t == 0)
    def _init():
        h_scr[...] = jnp.zeros_like(h_scr)

    h = h_scr[...]
    x_t = x_ref[0]
    gi = jnp.dot(x_t, wih_ref[...], preferred_element_type=jnp.float32)
    gi = gi + bih_ref[...]
    gh = jnp.dot(h, whh_ref[...], preferred_element_type=jnp.float32)
    gh = gh + bhh_ref[...]
    r = jax.nn.sigmoid(gi[:, :H] + gh[:, :H])
    z = jax.nn.sigmoid(gi[:, H:2 * H] + gh[:, H:2 * H])
    n = jnp.tanh(gi[:, 2 * H:] + r * gh[:, 2 * H:])
    h_new = (1.0 - z) * n + z * h
    valid = t < lens_ref[...]
    h_keep = jnp.where(valid, h_new, h)
    h_scr[...] = h_keep
    out_ref[0] = jnp.where(valid, h_new, 0.0)

    @pl.when(t == L - 1)
    def _fin():
        hid_ref[...] = h_keep


def _make_gru(B, L, E, H):
    return pl.pallas_call(
        functools.partial(_gru_body, L, H),
        grid=(L,),
        in_specs=[
            pl.BlockSpec((B, 1), lambda t: (0, 0)),
            pl.BlockSpec((E, 3 * H), lambda t: (0, 0)),
            pl.BlockSpec((H, 3 * H), lambda t: (0, 0)),
            pl.BlockSpec((1, 3 * H), lambda t: (0, 0)),
            pl.BlockSpec((1, 3 * H), lambda t: (0, 0)),
            pl.BlockSpec((1, B, E), lambda t: (t, 0, 0)),
        ],
        out_specs=[
            pl.BlockSpec((1, B, H), lambda t: (t, 0, 0)),
            pl.BlockSpec((B, H), lambda t: (0, 0)),
        ],
        out_shape=[
            jax.ShapeDtypeStruct((L, B, H), jnp.float32),
            jax.ShapeDtypeStruct((B, H), jnp.float32),
        ],
        scratch_shapes=[pltpu.VMEM((B, H), jnp.float32)],
    )


def kernel(input_seqs, seq_lengths, table, W_ih, W_hh, b_ih, b_hh):
    B, L = input_seqs.shape
    V, E = table.shape
    H = W_hh.shape[1]

    order = jnp.argsort(-seq_lengths)
    lengths = seq_lengths[order]
    seqs = input_seqs[order]

    info = plsc.get_sparse_core_info()
    NW = info.num_cores * info.num_subcores
    N = B * L
    per_w = N // NW
    CH = 80
    idx = jnp.transpose(seqs, (1, 0)).reshape(NW, per_w // CH, CH)

    emb = _make_sc_gather(V, E, N)(table, idx)
    x = emb.reshape(L, B, E)

    out_lbh, hT = _make_gru(B, L, E, H)(
        lengths[:, None],
        jnp.transpose(W_ih, (1, 0)),
        jnp.transpose(W_hh, (1, 0)),
        b_ih[None, :],
        b_hh[None, :],
        x,
    )
    outputs = jnp.transpose(out_lbh, (1, 0, 2))
    return outputs, hT[None, :, :]
